# baseline (speedup 1.0000x reference)
import functools

import jax
import jax.numpy as jnp
from jax import lax
from jax.experimental import pallas as pl
from jax.experimental.pallas import tpu as pltpu
from jax.experimental.pallas import tpu_sc as plsc

N = 1048576
NC = 2
NS = 16
L = 16
NW = NC * NS
PER_W = N // NW
C = 2048
NCHUNK = PER_W // C
NBUF = 2

_mesh = plsc.VectorSubcoreMesh(core_axis_name="c", subcore_axis_name="s")


def _buf_set():
    return [
        pltpu.VMEM((C,), jnp.int32),
        pltpu.VMEM((C,), jnp.float32),
        pltpu.VMEM((C,), jnp.float32),
        pltpu.VMEM((C,), jnp.float32),
        pltpu.VMEM((C,), jnp.float32),
        pltpu.VMEM((C,), jnp.float32),
        pltpu.VMEM((C,), jnp.float32),
        pltpu.VMEM((C,), jnp.float32),
        pltpu.VMEM((C,), jnp.float32),
        pltpu.VMEM((C,), jnp.float32),
        pltpu.SemaphoreType.DMA,
        pltpu.SemaphoreType.DMA,
        pltpu.SemaphoreType.DMA,
        pltpu.SemaphoreType.DMA,
        pltpu.SemaphoreType.DMA,
    ]


_BUFLEN = 15


@functools.partial(
    pl.kernel,
    mesh=_mesh,
    out_type=jax.ShapeDtypeStruct((NW, L), jnp.float32),
    scratch_types=[
        *_buf_set(),
        *_buf_set(),
        pltpu.VMEM((L,), jnp.float32),
    ],
)
def _rmse_partials(y_hbm, j_hbm, k1_hbm, k2_hbm, sub_hbm,
                   a_hbm, u_hbm, lam_hbm, g1_hbm, g2_hbm,
                   out_hbm, *scratch):
    bufs = [scratch[:_BUFLEN], scratch[_BUFLEN:2 * _BUFLEN]]
    acc_v = scratch[2 * _BUFLEN]
    wid = lax.axis_index("s") * NC + lax.axis_index("c")
    base = wid * PER_W

    tabs = (a_hbm, u_hbm, lam_hbm, g1_hbm, g2_hbm)

    def fire(c, buf):
        idx_v, y_v, j_v, k1_v, k2_v = buf[:5]
        gat = buf[5:10]
        sems = buf[10:15]
        off = base + c * C
        pltpu.sync_copy(sub_hbm.at[pl.ds(off, C)], idx_v)
        pltpu.sync_copy(y_hbm.at[pl.ds(off, C)], y_v)
        pltpu.sync_copy(j_hbm.at[pl.ds(off, C)], j_v)
        pltpu.sync_copy(k1_hbm.at[pl.ds(off, C)], k1_v)
        pltpu.sync_copy(k2_hbm.at[pl.ds(off, C)], k2_v)
        for t in range(5):
            pltpu.async_copy(tabs[t].at[idx_v], gat[t], sems[t])

    def drain(buf):
        idx_v = buf[0]
        gat = buf[5:10]
        sems = buf[10:15]
        for t in range(5):
            pltpu.make_async_copy(tabs[t].at[idx_v], gat[t], sems[t]).wait()

    def compute(buf, acc):
        _, y_v, j_v, k1_v, k2_v = buf[:5]
        a_v, u_v, lam_v, g1_v, g2_v = buf[5:10]

        def vec_body(v, acc16):
            s = pl.ds(v * L, L)
            a = jnp.maximum(a_v[s], 0.0)
            u = jnp.maximum(u_v[s], 0.0)
            lm = 0.2 / (1.0 + jnp.exp(-lam_v[s]))
            g1 = 1.0 / (1.0 + jnp.exp(-g1_v[s]))
            g2 = 1.0 / (1.0 + jnp.exp(-g2_v[s]))
            t = j_v[s] + g1 * k1_v[s] + g2 * k2_v[s]
            mu = a - u * jnp.exp(-lm * t)
            resid = y_v[s] - mu
            return acc16 + resid * resid

        return lax.fori_loop(0, C // L, vec_body, acc)

    acc = jnp.zeros((L,), jnp.float32)
    fire(0, bufs[0])
    for c in range(NCHUNK):
        cur = bufs[c % NBUF]
        if c + 1 < NCHUNK:
            fire(c + 1, bufs[(c + 1) % NBUF])
        drain(cur)
        acc = compute(cur, acc)
    acc_v[...] = acc
    pltpu.sync_copy(acc_v, out_hbm.at[wid])


def kernel(y, j, k1, k2, sub, A, U, Lambda, Gamma1, Gamma2):
    partials = _rmse_partials(y, j, k1, k2, sub.astype(jnp.int32),
                              A, U, Lambda, Gamma1, Gamma2)
    return jnp.sqrt(jnp.sum(partials) / N)


# speedup vs baseline: 1.0563x; 1.0563x over previous
import functools

import jax
import jax.numpy as jnp
from jax import lax
from jax.experimental import pallas as pl
from jax.experimental.pallas import tpu as pltpu
from jax.experimental.pallas import tpu_sc as plsc

N = 1048576
NC = 2
NS = 16
L = 16
NW = NC * NS
PER_W = N // NW
C = 4096
UNROLL = 4
NCHUNK = PER_W // C
NBUF = 2

_mesh = plsc.VectorSubcoreMesh(core_axis_name="c", subcore_axis_name="s")


def _buf_set():
    return [
        pltpu.VMEM((C,), jnp.int32),
        pltpu.VMEM((C,), jnp.float32),
        pltpu.VMEM((C,), jnp.float32),
        pltpu.VMEM((C,), jnp.float32),
        pltpu.VMEM((C,), jnp.float32),
        pltpu.VMEM((C,), jnp.float32),
        pltpu.VMEM((C,), jnp.float32),
        pltpu.VMEM((C,), jnp.float32),
        pltpu.VMEM((C,), jnp.float32),
        pltpu.VMEM((C,), jnp.float32),
        pltpu.SemaphoreType.DMA,
        pltpu.SemaphoreType.DMA,
        pltpu.SemaphoreType.DMA,
        pltpu.SemaphoreType.DMA,
        pltpu.SemaphoreType.DMA,
    ]


_BUFLEN = 15


@functools.partial(
    pl.kernel,
    mesh=_mesh,
    out_type=jax.ShapeDtypeStruct((NW, L), jnp.float32),
    scratch_types=[
        *_buf_set(),
        *_buf_set(),
        pltpu.VMEM((L,), jnp.float32),
    ],
)
def _rmse_partials(y_hbm, j_hbm, k1_hbm, k2_hbm, sub_hbm,
                   a_hbm, u_hbm, lam_hbm, g1_hbm, g2_hbm,
                   out_hbm, *scratch):
    bufs = [scratch[:_BUFLEN], scratch[_BUFLEN:2 * _BUFLEN]]
    acc_v = scratch[2 * _BUFLEN]
    wid = lax.axis_index("s") * NC + lax.axis_index("c")
    base = wid * PER_W

    tabs = (a_hbm, u_hbm, lam_hbm, g1_hbm, g2_hbm)

    def fire(c, buf):
        idx_v, y_v, j_v, k1_v, k2_v = buf[:5]
        gat = buf[5:10]
        sems = buf[10:15]
        off = base + c * C
        pltpu.sync_copy(sub_hbm.at[pl.ds(off, C)], idx_v)
        pltpu.sync_copy(y_hbm.at[pl.ds(off, C)], y_v)
        pltpu.sync_copy(j_hbm.at[pl.ds(off, C)], j_v)
        pltpu.sync_copy(k1_hbm.at[pl.ds(off, C)], k1_v)
        pltpu.sync_copy(k2_hbm.at[pl.ds(off, C)], k2_v)
        for t in range(5):
            pltpu.async_copy(tabs[t].at[idx_v], gat[t], sems[t])

    def drain(buf):
        idx_v = buf[0]
        gat = buf[5:10]
        sems = buf[10:15]
        for t in range(5):
            pltpu.make_async_copy(tabs[t].at[idx_v], gat[t], sems[t]).wait()

    def compute(buf, acc):
        _, y_v, j_v, k1_v, k2_v = buf[:5]
        a_v, u_v, lam_v, g1_v, g2_v = buf[5:10]

        def vec_body(v, accs):
            out = []
            for k in range(UNROLL):
                s = pl.ds((v * UNROLL + k) * L, L)
                a = jnp.maximum(a_v[s], 0.0)
                u = jnp.maximum(u_v[s], 0.0)
                lm = 0.2 / (1.0 + jnp.exp(-lam_v[s]))
                g1 = 1.0 / (1.0 + jnp.exp(-g1_v[s]))
                g2 = 1.0 / (1.0 + jnp.exp(-g2_v[s]))
                t = j_v[s] + g1 * k1_v[s] + g2 * k2_v[s]
                mu = a - u * jnp.exp(-lm * t)
                resid = y_v[s] - mu
                out.append(accs[k] + resid * resid)
            return tuple(out)

        return lax.fori_loop(0, C // (L * UNROLL), vec_body, acc)

    acc = tuple(jnp.zeros((L,), jnp.float32) for _ in range(UNROLL))
    fire(0, bufs[0])
    for c in range(NCHUNK):
        cur = bufs[c % NBUF]
        if c + 1 < NCHUNK:
            fire(c + 1, bufs[(c + 1) % NBUF])
        drain(cur)
        acc = compute(cur, acc)
    total = acc[0]
    for k in range(1, UNROLL):
        total = total + acc[k]
    acc_v[...] = total
    pltpu.sync_copy(acc_v, out_hbm.at[wid])


def kernel(y, j, k1, k2, sub, A, U, Lambda, Gamma1, Gamma2):
    partials = _rmse_partials(y, j, k1, k2, sub.astype(jnp.int32),
                              A, U, Lambda, Gamma1, Gamma2)
    return jnp.sqrt(jnp.sum(partials) / N)


# speedup vs baseline: 1.0917x; 1.0336x over previous
import functools

import jax
import jax.numpy as jnp
from jax import lax
from jax.experimental import pallas as pl
from jax.experimental.pallas import tpu as pltpu
from jax.experimental.pallas import tpu_sc as plsc

N = 1048576
NC = 2
NS = 16
L = 16
NW = NC * NS
PER_W = N // NW
C = 4096
UNROLL = 4
NCHUNK = PER_W // C
NBUF = 3

_mesh = plsc.VectorSubcoreMesh(core_axis_name="c", subcore_axis_name="s")


def _buf_set():
    return [
        pltpu.VMEM((C,), jnp.int32),
        pltpu.VMEM((C,), jnp.float32),
        pltpu.VMEM((C,), jnp.float32),
        pltpu.VMEM((C,), jnp.float32),
        pltpu.VMEM((C,), jnp.float32),
        pltpu.VMEM((C,), jnp.float32),
        pltpu.VMEM((C,), jnp.float32),
        pltpu.VMEM((C,), jnp.float32),
        pltpu.VMEM((C,), jnp.float32),
        pltpu.VMEM((C,), jnp.float32),
        pltpu.SemaphoreType.DMA,
        pltpu.SemaphoreType.DMA,
        pltpu.SemaphoreType.DMA,
        pltpu.SemaphoreType.DMA,
        pltpu.SemaphoreType.DMA,
        pltpu.SemaphoreType.DMA,
    ]


_BUFLEN = 16


@functools.partial(
    pl.kernel,
    mesh=_mesh,
    out_type=jax.ShapeDtypeStruct((NW, L), jnp.float32),
    scratch_types=[
        *_buf_set(),
        *_buf_set(),
        *_buf_set(),
        pltpu.VMEM((L,), jnp.float32),
    ],
)
def _rmse_partials(y_hbm, j_hbm, k1_hbm, k2_hbm, sub_hbm,
                   a_hbm, u_hbm, lam_hbm, g1_hbm, g2_hbm,
                   out_hbm, *scratch):
    bufs = [scratch[i * _BUFLEN:(i + 1) * _BUFLEN] for i in range(NBUF)]
    acc_v = scratch[NBUF * _BUFLEN]
    wid = lax.axis_index("s") * NC + lax.axis_index("c")
    base = wid * PER_W

    tabs = (a_hbm, u_hbm, lam_hbm, g1_hbm, g2_hbm)

    def fire(c, buf):
        idx_v, y_v, j_v, k1_v, k2_v = buf[:5]
        gat = buf[5:10]
        sems = buf[10:15]
        osem = buf[15]
        off = base + c * C
        pltpu.sync_copy(sub_hbm.at[pl.ds(off, C)], idx_v)
        for t in range(5):
            pltpu.async_copy(tabs[t].at[idx_v], gat[t], sems[t])
        pltpu.async_copy(y_hbm.at[pl.ds(off, C)], y_v, osem)
        pltpu.async_copy(j_hbm.at[pl.ds(off, C)], j_v, osem)
        pltpu.async_copy(k1_hbm.at[pl.ds(off, C)], k1_v, osem)
        pltpu.async_copy(k2_hbm.at[pl.ds(off, C)], k2_v, osem)

    def drain(c, buf):
        idx_v, y_v, j_v, k1_v, k2_v = buf[:5]
        gat = buf[5:10]
        sems = buf[10:15]
        osem = buf[15]
        off = base + c * C
        for t in range(5):
            pltpu.make_async_copy(tabs[t].at[idx_v], gat[t], sems[t]).wait()
        pltpu.make_async_copy(y_hbm.at[pl.ds(off, C)], y_v, osem).wait()
        pltpu.make_async_copy(j_hbm.at[pl.ds(off, C)], j_v, osem).wait()
        pltpu.make_async_copy(k1_hbm.at[pl.ds(off, C)], k1_v, osem).wait()
        pltpu.make_async_copy(k2_hbm.at[pl.ds(off, C)], k2_v, osem).wait()

    def compute(buf, acc):
        _, y_v, j_v, k1_v, k2_v = buf[:5]
        a_v, u_v, lam_v, g1_v, g2_v = buf[5:10]

        def vec_body(v, accs):
            out = []
            for k in range(UNROLL):
                s = pl.ds((v * UNROLL + k) * L, L)
                a = jnp.maximum(a_v[s], 0.0)
                u = jnp.maximum(u_v[s], 0.0)
                lm = 0.2 / (1.0 + jnp.exp(-lam_v[s]))
                g1 = 1.0 / (1.0 + jnp.exp(-g1_v[s]))
                g2 = 1.0 / (1.0 + jnp.exp(-g2_v[s]))
                t = j_v[s] + g1 * k1_v[s] + g2 * k2_v[s]
                mu = a - u * jnp.exp(-lm * t)
                resid = y_v[s] - mu
                out.append(accs[k] + resid * resid)
            return tuple(out)

        return lax.fori_loop(0, C // (L * UNROLL), vec_body, acc)

    acc = tuple(jnp.zeros((L,), jnp.float32) for _ in range(UNROLL))
    for c in range(NBUF - 1):
        fire(c, bufs[c])
    for c in range(NCHUNK):
        cur = bufs[c % NBUF]
        if c + NBUF - 1 < NCHUNK:
            fire(c + NBUF - 1, bufs[(c + NBUF - 1) % NBUF])
        drain(c, cur)
        acc = compute(cur, acc)
    total = acc[0]
    for k in range(1, UNROLL):
        total = total + acc[k]
    acc_v[...] = total
    pltpu.sync_copy(acc_v, out_hbm.at[wid])


def kernel(y, j, k1, k2, sub, A, U, Lambda, Gamma1, Gamma2):
    partials = _rmse_partials(y, j, k1, k2, sub.astype(jnp.int32),
                              A, U, Lambda, Gamma1, Gamma2)
    return jnp.sqrt(jnp.sum(partials) / N)
